# baseline (speedup 1.0000x reference)
import functools

import jax
import jax.numpy as jnp
from jax import lax
from jax.experimental import pallas as pl
from jax.experimental.pallas import tpu as pltpu
from jax.experimental.pallas import tpu_sc as plsc

PAD = 0
D = 64
LANES = 16
NC, NS = 2, 16
NW = NC * NS
C = 256
V = 1000000

NBLK = (V + 127) // 128
BLK_EVEN = (NBLK // NW) * NW
BLK_PER_W = BLK_EVEN // NW
N_EXTRA = NBLK - BLK_EVEN


def _mesh():
    return plsc.VectorSubcoreMesh(core_axis_name="c", subcore_axis_name="s",
                                  num_cores=NC, num_subcores=NS)


@functools.lru_cache(maxsize=None)
def _detranspose_call():
    @functools.partial(
        pl.kernel,
        out_type=jax.ShapeDtypeStruct((V * D // 128, 128), jnp.float32),
        mesh=_mesh(),
        compiler_params=pltpu.CompilerParams(needs_layout_passes=False),
        scratch_types=[
            pltpu.VMEM((2, 64, 128), jnp.float32),
            pltpu.VMEM((2, 64, 128), jnp.float32),
            pltpu.SemaphoreType.DMA,
            pltpu.SemaphoreType.DMA,
            pltpu.SemaphoreType.DMA,
            pltpu.SemaphoreType.DMA,
        ],
    )
    def runT(tt_hbm, out_hbm, src_v, dst_v, si0, si1, so0, so1):
        si = (si0, si1)
        so = (so0, so1)
        wid = lax.axis_index("s") * NC + lax.axis_index("c")
        blk0 = wid * BLK_PER_W

        iota = lax.iota(jnp.int32, 16)
        rvecs = [k * LANES + iota for k in range(D // LANES)]

        def fire_in(blk, b):
            pltpu.async_copy(tt_hbm.at[:, pl.ds(blk * 128, 128)],
                             src_v.at[b], si[b])

        def wait_in(blk, b):
            pltpu.make_async_copy(tt_hbm.at[:, pl.ds(blk * 128, 128)],
                                  src_v.at[b], si[b]).wait()

        def compute(b):
            def col(cp, carry):
                row = cp // 2
                cbase = (cp % 2) * D
                cvec = jnp.full((LANES,), cp, jnp.int32)
                for k in range(D // LANES):
                    g = plsc.load_gather(src_v.at[b], [rvecs[k], cvec])
                    dst_v[b, row, pl.ds(cbase + k * LANES, LANES)] = g
                return carry

            lax.fori_loop(0, 128, col, 0)

        def fire_out(blk, b):
            pltpu.async_copy(dst_v.at[b], out_hbm.at[pl.ds(blk * 64, 64)],
                             so[b])

        def wait_out(blk, b):
            pltpu.make_async_copy(dst_v.at[b],
                                  out_hbm.at[pl.ds(blk * 64, 64)],
                                  so[b]).wait()

        fire_in(blk0 + 0, 0)
        fire_in(blk0 + 1, 1)

        def pair(j0, carry):
            for b in range(2):
                j = j0 * 2 + b
                blk = blk0 + j
                wait_in(blk, b)

                @pl.when(j0 >= 1)
                def _():
                    wait_out(blk - 2, b)

                compute(b)
                fire_out(blk, b)

                @pl.when(j + 2 < BLK_PER_W)
                def _():
                    fire_in(blk + 2, b)

            return carry

        lax.fori_loop(0, BLK_PER_W // 2, pair, 0)
        wait_out(blk0 + BLK_PER_W - 2, 0)
        wait_out(blk0 + BLK_PER_W - 1, 1)

        @pl.when(wid < N_EXTRA)
        def _():
            blk = BLK_EVEN + wid
            fire_in(blk, 0)
            wait_in(blk, 0)
            compute(0)

            @pl.when(blk < NBLK - 1)
            def _():
                fire_out(blk, 0)
                wait_out(blk, 0)

            @pl.when(blk == NBLK - 1)
            def _():
                pltpu.sync_copy(dst_v.at[0, pl.ds(0, 32)],
                                out_hbm.at[pl.ds(blk * 64, 32)])

    return runT


@functools.lru_cache(maxsize=None)
def _emb_call(B, J):
    nbb = B // 128
    nblocks = (B * J) // C
    per_w = nblocks // NW
    blk_j = B // C
    rows_dd = nbb * 8
    assert B % C == 0 and nblocks % NW == 0 and C == 2 * 128

    @functools.partial(
        pl.kernel,
        out_type=jax.ShapeDtypeStruct((B * J // 2, 128), jnp.float32),
        mesh=_mesh(),
        compiler_params=pltpu.CompilerParams(use_tc_tiling_on_sc=False,
                                             needs_layout_passes=False),
        scratch_types=[
            pltpu.VMEM((per_w * C,), jnp.int32),
            pltpu.VMEM((2, C, D), jnp.float32),
            pltpu.VMEM((2, D // 8, 16, 128), jnp.float32),
            pltpu.SemaphoreType.DMA,
            pltpu.SemaphoreType.DMA,
            pltpu.SemaphoreType.DMA,
            pltpu.SemaphoreType.DMA,
        ],
    )
    def run(ids_hbm, table_hbm, out_hbm, idx_v, src_v, dst_v,
            sg0, sg1, sw0, sw1):
        sg = (sg0, sg1)
        sw = (sw0, sw1)
        wid = lax.axis_index("s") * NC + lax.axis_index("c")
        t0 = wid * per_w

        iota = lax.iota(jnp.int32, LANES)
        lvecs = [bb * 128 + g * LANES + iota
                 for bb in range(2) for g in range(128 // LANES)]

        pltpu.sync_copy(ids_hbm.at[pl.ds(t0 * C, per_w * C)], idx_v)

        def fire_gather(tl, b):
            pltpu.async_copy(table_hbm.at[idx_v.at[pl.ds(tl * C, C)]],
                             src_v.at[b], sg[b])

        def wait_gather(tl, b):
            pltpu.make_async_copy(table_hbm.at[idx_v.at[pl.ds(tl * C, C)]],
                                  src_v.at[b], sg[b]).wait()

        def row0(tl, dd):
            t = t0 + tl
            j = t // blk_j
            bbg = t % blk_j
            return j * (8 * rows_dd) + dd * rows_dd + bbg * 16

        def fire_writes(tl, b):
            for dd in range(D // 8):
                pltpu.async_copy(dst_v.at[b, dd],
                                 out_hbm.at[pl.ds(row0(tl, dd), 16)], sw[b])

        def wait_writes(tl, b):
            for dd in range(D // 8):
                pltpu.make_async_copy(
                    dst_v.at[b, dd],
                    out_hbm.at[pl.ds(row0(tl, dd), 16)], sw[b]).wait()

        def transpose(tl, b):
            scales = []
            for u in range(C // LANES):
                idv = idx_v[pl.ds(tl * C + u * LANES, LANES)]
                scales.append(jnp.minimum(idv, 1).astype(jnp.float32))

            def body(dd, carry):
                for bb in range(2):
                    for d8 in range(8):
                        dvec = jnp.full((LANES,), dd * 8 + d8, jnp.int32)
                        for g in range(128 // LANES):
                            u = bb * 8 + g
                            val = plsc.load_gather(src_v.at[b],
                                                   [lvecs[u], dvec])
                            dst_v[b, dd, bb * 8 + d8,
                                  pl.ds(g * LANES, LANES)] = val * scales[u]
                return carry

            lax.fori_loop(0, D // 8, body, 0)

        fire_gather(0, 0)
        fire_gather(1, 1)

        def do_block(tl, b, first):
            wait_gather(tl, b)
            if not first:
                wait_writes(tl - 2, b)
            transpose(tl, b)
            fire_writes(tl, b)

            @pl.when(tl + 2 < per_w)
            def _():
                fire_gather(tl + 2, b)

        do_block(0, 0, True)
        do_block(1, 1, True)

        def pair(i, carry):
            tl = 2 + 2 * i
            do_block(tl, 0, False)
            do_block(tl + 1, 1, False)
            return carry

        lax.fori_loop(0, (per_w - 2) // 2, pair, 0)
        wait_writes(per_w - 2, 0)
        wait_writes(per_w - 1, 1)

    return run


def kernel(ids, table):
    B, J = ids.shape
    packed = _detranspose_call()(table.T)
    tlin = packed.reshape(V, D)
    ids_lin = ids.T.reshape(B * J).astype(jnp.int32)
    w = _emb_call(B, J)(ids_lin, tlin)
    return (w.reshape(J, 8, B // 128, 8, 128)
             .transpose(2, 4, 0, 1, 3)
             .reshape(B, J, D))


# speedup vs baseline: 2.3682x; 2.3682x over previous
import functools

import jax
import jax.numpy as jnp
from jax import lax
from jax.experimental import pallas as pl
from jax.experimental.pallas import tpu as pltpu
from jax.experimental.pallas import tpu_sc as plsc

PAD = 0
D = 64
LANES = 16
NC, NS = 2, 16
NW = NC * NS
C = 512
V = 1000000


def _scalar_min16(vec):
    m = vec[0]
    for u in range(1, LANES):
        m = jnp.minimum(m, vec[u])
    return m


def _mesh():
    return plsc.VectorSubcoreMesh(core_axis_name="c", subcore_axis_name="s",
                                  num_cores=NC, num_subcores=NS)


@functools.lru_cache(maxsize=None)
def _emb_call(B):
    per_w = B // NW
    nch = per_w // C
    assert B % NW == 0 and per_w % C == 0
    assert nch >= 4 and nch % 2 == 0

    @functools.partial(
        pl.kernel,
        out_type=jax.ShapeDtypeStruct((B, D), jnp.float32),
        mesh=_mesh(),
        compiler_params=pltpu.CompilerParams(use_tc_tiling_on_sc=False),
        scratch_types=[
            pltpu.VMEM((per_w,), jnp.int32),
            pltpu.VMEM((2, C, D), jnp.float32),
            pltpu.SemaphoreType.DMA,
            pltpu.SemaphoreType.DMA,
            pltpu.SemaphoreType.DMA,
            pltpu.SemaphoreType.DMA,
        ],
    )
    def run(ids_hbm, table_hbm, out_hbm, idx_v, rows_v, sg0, sg1, sw0, sw1):
        sg = (sg0, sg1)
        sw = (sw0, sw1)
        wid = lax.axis_index("s") * NC + lax.axis_index("c")
        base0 = wid * per_w

        pltpu.sync_copy(ids_hbm.at[pl.ds(base0, per_w)], idx_v)

        def fire_chunk(j, b):
            pltpu.async_copy(table_hbm.at[idx_v.at[pl.ds(j * C, C)]],
                             rows_v.at[b], sg[b])

        def wait_gather(j, b):
            pltpu.make_async_copy(table_hbm.at[idx_v.at[pl.ds(j * C, C)]],
                                  rows_v.at[b], sg[b]).wait()

        def fire_write(j, b):
            pltpu.async_copy(rows_v.at[b],
                             out_hbm.at[pl.ds(base0 + j * C, C)], sw[b])

        def wait_write(j, b):
            pltpu.make_async_copy(rows_v.at[b],
                                  out_hbm.at[pl.ds(base0 + j * C, C)],
                                  sw[b]).wait()

        def fix(j, b):
            def gmin(g, acc):
                return jnp.minimum(acc, idx_v[pl.ds(j * C + g * LANES, LANES)])

            acc = lax.fori_loop(0, C // LANES, gmin,
                                jnp.full((LANES,), jnp.iinfo(jnp.int32).max,
                                         jnp.int32))

            @pl.when(_scalar_min16(acc) == PAD)
            def _():
                def gfix(g, carry):
                    ivec = idx_v[pl.ds(j * C + g * LANES, LANES)]

                    @pl.when(_scalar_min16(ivec) == PAD)
                    def _():
                        for u in range(LANES):
                            scale = jnp.where(ivec[u] == PAD, 0.0, 1.0)
                            for cc in range(D // LANES):
                                sl = pl.ds(cc * LANES, LANES)
                                rows_v[b, g * LANES + u, sl] = (
                                    rows_v[b, g * LANES + u, sl] * scale)

                    return carry

                lax.fori_loop(0, C // LANES, gfix, 0)

        fire_chunk(0, 0)
        wait_gather(0, 0)
        fix(0, 0)
        fire_chunk(1, 1)
        fire_write(0, 0)

        def main_pair(i, carry):
            j = 1 + 2 * i
            for b, dj in ((1, 0), (0, 1)):
                jj = j + dj
                wait_gather(jj, b)
                fix(jj, b)
                fire_write(jj, b)
                wait_write(jj - 1, 1 - b)
                fire_chunk(jj + 1, 1 - b)
            return carry

        lax.fori_loop(0, (nch - 2) // 2, main_pair, 0)

        wait_gather(nch - 1, 1)
        fix(nch - 1, 1)
        fire_write(nch - 1, 1)
        wait_write(nch - 2, 0)
        wait_write(nch - 1, 1)

    return run


def kernel(ids, table):
    shp = ids.shape
    B = ids.size
    out = _emb_call(B)(ids.reshape(B).astype(jnp.int32), table)
    return out.reshape(*shp, D)
